# baseline (speedup 1.0000x reference)
import functools

import jax
import jax.numpy as jnp
from jax import lax
from jax.experimental import pallas as pl
from jax.experimental.pallas import tpu as pltpu
from jax.experimental.pallas import tpu_sc as plsc

N = 10000
E = 320000
D = 128

NC = 2
NS = 16
NW = NC * NS

CH = 128
NCH = 160
E_PAD = NS * CH * NCH
NPAD = 10240
RPT = NPAD // NS


W = 8
PAIRS = NCH // (2 * W)


def _sc_body(with_counts, *refs):
    if with_counts:
        (y_hbm, src_hbm, dst_hbm, out_hbm, cnt_hbm,
         sA, dA, sB, dB, rows0, rows1, ones_v, zc_v,
         acc_sh, cnt_sh, *sems) = refs
    else:
        (y_hbm, src_hbm, dst_hbm, out_hbm,
         sA, dA, sB, dB, rows0, rows1, ones_v, zc_v,
         acc_sh, cnt_sh, *sems) = refs
    rows = (rows0, rows1)
    gs0, gs1, ss0, ss1, cs0, cs1, iwA, iwB = sems
    gs = (gs0, gs1)
    ss = (ss0, ss1)
    cs = (cs0, cs1)

    s = lax.axis_index("s")
    base = s * RPT
    tile_chunk0 = s * NCH

    zeros16 = jnp.zeros((16,), jnp.float32)

    def _zrow(i, carry):
        for j in range(D // 16):
            rows0[i, pl.ds(j * 16, 16)] = zeros16
        return carry
    lax.fori_loop(0, CH, _zrow, 0)
    for j in range(CH // 16):
        ones_v[pl.ds(j * 16, 16)] = jnp.ones((16,), jnp.float32)
    for j in range(RPT // 16):
        zc_v[pl.ds(j * 16, 16)] = zeros16

    for k in range(RPT // CH):
        pltpu.sync_copy(rows0, acc_sh.at[pl.ds(base + k * CH, CH)])
    pltpu.sync_copy(zc_v, cnt_sh.at[pl.ds(base, RPT)])

    def _fetch(win_idx, sbuf, dbuf, sem):
        off = tile_chunk0 + win_idx * W
        pltpu.async_copy(src_hbm.at[pl.ds(off, W)], sbuf, sem)
        pltpu.async_copy(dst_hbm.at[pl.ds(off, W)], dbuf, sem)

    def _fetch_wait(sbuf, dbuf, sem):
        pltpu.make_async_copy(src_hbm.at[pl.ds(0, W)], sbuf, sem).wait()
        pltpu.make_async_copy(dst_hbm.at[pl.ds(0, W)], dbuf, sem).wait()

    _fetch(0, sA, dA, iwA)
    _fetch(1, sB, dB, iwB)
    _fetch_wait(sA, dA, iwA)
    pltpu.async_copy(y_hbm.at[sA.at[0]], rows[0], gs[0])
    pltpu.async_copy(y_hbm.at[sA.at[1]], rows[1], gs[1])

    plsc.subcore_barrier()

    def _pair(i, carry):
        not_last = i < PAIRS - 1
        for phase in range(2):
            swin, dwin = (sA, dA) if phase == 0 else (sB, dB)
            nswin = sB if phase == 0 else sA
            for kk in range(W):
                p = kk % 2
                if kk == 6:
                    if phase == 0:
                        _fetch_wait(sB, dB, iwB)
                    else:
                        @pl.when(not_last)
                        def _():
                            _fetch_wait(sA, dA, iwA)
                pltpu.make_async_copy(y_hbm.at[swin.at[kk]], rows[p], gs[p]).wait()
                pltpu.async_copy(rows[p], acc_sh.at[dwin.at[kk]], ss[p], add=True)
                if with_counts:
                    pltpu.async_copy(ones_v, cnt_sh.at[dwin.at[kk]], cs[p], add=True)
                pltpu.make_async_copy(rows[p], acc_sh.at[dwin.at[kk]], ss[p]).wait()
                if kk < W - 2:
                    pltpu.async_copy(y_hbm.at[swin.at[kk + 2]], rows[p], gs[p])
                else:
                    lookahead_ok = (phase == 0) or not_last

                    def _issue(nref=nswin, row=rows[p], sem=gs[p], kkn=kk - 6):
                        pltpu.async_copy(y_hbm.at[nref.at[kkn]], row, sem)
                    if lookahead_ok is True:
                        _issue()
                    else:
                        pl.when(not_last)(_issue)
                if with_counts:
                    pltpu.make_async_copy(ones_v, cnt_sh.at[dwin.at[kk]], cs[p]).wait()
            @pl.when(not_last)
            def _():
                if phase == 0:
                    _fetch(2 * i + 2, sA, dA, iwA)
                else:
                    _fetch(2 * i + 3, sB, dB, iwB)
        return carry
    lax.fori_loop(0, PAIRS, _pair, 0)

    plsc.subcore_barrier()

    pltpu.sync_copy(acc_sh.at[pl.ds(base, RPT)], out_hbm.at[pl.ds(base, RPT)])
    if with_counts:
        pltpu.sync_copy(cnt_sh.at[pl.ds(base, RPT)],
                        cnt_hbm.at[pl.ds(base, RPT)])


def _make_sc_kernel(with_counts):
    out_type = [jax.ShapeDtypeStruct((NPAD, D), jnp.float32)]
    if with_counts:
        out_type.append(jax.ShapeDtypeStruct((NPAD,), jnp.float32))
    mesh = plsc.VectorSubcoreMesh(core_axis_name="c", subcore_axis_name="s",
                                  num_cores=1)
    return pl.kernel(
        functools.partial(_sc_body, with_counts),
        out_type=out_type,
        mesh=mesh,
        scratch_types=[
            pltpu.VMEM((W, CH), jnp.int32),
            pltpu.VMEM((W, CH), jnp.int32),
            pltpu.VMEM((W, CH), jnp.int32),
            pltpu.VMEM((W, CH), jnp.int32),
            pltpu.VMEM((CH, D), jnp.float32),
            pltpu.VMEM((CH, D), jnp.float32),
            pltpu.VMEM((CH,), jnp.float32),
            pltpu.VMEM((RPT,), jnp.float32),
            pltpu.VMEM_SHARED((NPAD, D), jnp.float32),
            pltpu.VMEM_SHARED((NPAD,), jnp.float32),
        ] + [pltpu.SemaphoreType.DMA] * 8,
        name="sage_segment_sum" + ("_cnt" if with_counts else ""),
    )


_sc_layer1 = _make_sc_kernel(True)
_sc_layer2 = _make_sc_kernel(False)


_BLK = 2000


def _dotT(a, w):
    return lax.dot_general(a, w, (((1,), (1,)), ((), ())),
                           preferred_element_type=jnp.float32)


def _pre_body(x_ref, wl_ref, wr_ref, b_ref, y_ref, z_ref):
    xb = x_ref[...]
    y_ref[...] = _dotT(xb, wl_ref[...])
    z_ref[...] = _dotT(xb, wr_ref[...]) + b_ref[...]


def _mid_body(p_ref, c_ref, z1_ref, wl_ref, wr_ref, b_ref, y2_ref, z2_ref):
    cnt = jnp.maximum(c_ref[...], 1.0)
    h = jnp.maximum(p_ref[...] / cnt + z1_ref[...], 0.0)
    y2_ref[...] = _dotT(h, wl_ref[...])
    z2_ref[...] = _dotT(h, wr_ref[...]) + b_ref[...]


def _post_body(p_ref, c_ref, z2_ref, o_ref):
    cnt = jnp.maximum(c_ref[...], 1.0)
    o_ref[...] = p_ref[...] / cnt + z2_ref[...]


_row_spec = pl.BlockSpec((_BLK, D), lambda i: (i, 0))
_cnt_spec = pl.BlockSpec((_BLK, 1), lambda i: (i, 0))
_w_spec = pl.BlockSpec((D, D), lambda i: (0, 0))
_b_spec = pl.BlockSpec((1, D), lambda i: (0, 0))
_f32 = jnp.float32


_pre_call = pl.pallas_call(
    _pre_body,
    grid=(N // _BLK,),
    in_specs=[_row_spec, _w_spec, _w_spec, _b_spec],
    out_specs=[_row_spec, _row_spec],
    out_shape=[jax.ShapeDtypeStruct((N, D), _f32)] * 2,
)

_mid_call = pl.pallas_call(
    _mid_body,
    grid=(N // _BLK,),
    in_specs=[_row_spec, _cnt_spec, _row_spec, _w_spec, _w_spec, _b_spec],
    out_specs=[_row_spec, _row_spec],
    out_shape=[jax.ShapeDtypeStruct((N, D), _f32)] * 2,
)

_post_call = pl.pallas_call(
    _post_body,
    grid=(N // _BLK,),
    in_specs=[_row_spec, _cnt_spec, _row_spec],
    out_specs=_row_spec,
    out_shape=jax.ShapeDtypeStruct((N, D), _f32),
)


def kernel(x, edge_index, W1l, b1, W1r, W2l, b2, W2r):
    src = edge_index[0].astype(jnp.int32)
    dst = edge_index[1].astype(jnp.int32)
    pad = E_PAD - E
    src_p = jnp.concatenate([src, jnp.zeros((pad,), jnp.int32)]).reshape(E_PAD // CH, CH)
    pad_dst = N + (jnp.arange(pad, dtype=jnp.int32) % (NPAD - N))
    dst_p = jnp.concatenate([dst, pad_dst]).reshape(E_PAD // CH, CH)

    b1r = b1.reshape(1, D)
    b2r = b2.reshape(1, D)

    y1, z1 = _pre_call(x, W1l, W1r, b1r)
    agg1, cnts = _sc_layer1(y1, src_p, dst_p)
    cn = cnts[0:N].reshape(N, 1)

    y2, z2 = _mid_call(agg1[0:N], cn, z1, W2l, W2r, b2r)
    (agg2,) = _sc_layer2(y2, src_p, dst_p)
    return _post_call(agg2[0:N], cn, z2)


# speedup vs baseline: 3.3306x; 3.3306x over previous
import functools

import jax
import jax.numpy as jnp
from jax import lax
from jax.experimental import pallas as pl
from jax.experimental.pallas import tpu as pltpu
from jax.experimental.pallas import tpu_sc as plsc

N = 10000
E = 320000
D = 128

NC = 2
NS = 16
NW = NC * NS

CH = 128
NCH = 80
E_PAD = NW * CH * NCH
NREAL = E // CH
NPAD = 10240
RPT = NPAD // NS


W = 8
PAIRS = NCH // (2 * W)


def _sc_body(with_counts, *refs):
    if with_counts:
        (y_hbm, src_hbm, dst_hbm, out_hbm, cnt_hbm,
         sA, dA, sB, dB, rows0, rows1, ones_v, zc_v,
         acc_sh, cnt_sh, *sems) = refs
    else:
        (y_hbm, src_hbm, dst_hbm, out_hbm,
         sA, dA, sB, dB, rows0, rows1, ones_v, zc_v,
         acc_sh, cnt_sh, *sems) = refs
    rows = (rows0, rows1)
    gs0, gs1, ss0, ss1, cs0, cs1, iwA, iwB = sems
    gs = (gs0, gs1)
    ss = (ss0, ss1)
    cs = (cs0, cs1)

    c = lax.axis_index("c")
    s = lax.axis_index("s")
    base = s * RPT
    wid = c * NS + s
    tile_chunk0 = wid * NCH

    zeros16 = jnp.zeros((16,), jnp.float32)

    def _zrow(i, carry):
        for j in range(D // 16):
            rows0[i, pl.ds(j * 16, 16)] = zeros16
        return carry
    lax.fori_loop(0, CH, _zrow, 0)
    for j in range(CH // 16):
        ones_v[pl.ds(j * 16, 16)] = jnp.ones((16,), jnp.float32)
    for j in range(RPT // 16):
        zc_v[pl.ds(j * 16, 16)] = zeros16

    for k in range(RPT // CH):
        pltpu.sync_copy(rows0, acc_sh.at[pl.ds(base + k * CH, CH)])
    pltpu.sync_copy(zc_v, cnt_sh.at[pl.ds(base, RPT)])

    def _fetch(win_idx, sbuf, dbuf, sem):
        off = tile_chunk0 + win_idx * W
        pltpu.async_copy(src_hbm.at[pl.ds(off, W)], sbuf, sem)
        pltpu.async_copy(dst_hbm.at[pl.ds(off, W)], dbuf, sem)

    def _fetch_wait(sbuf, dbuf, sem):
        pltpu.make_async_copy(src_hbm.at[pl.ds(0, W)], sbuf, sem).wait()
        pltpu.make_async_copy(dst_hbm.at[pl.ds(0, W)], dbuf, sem).wait()

    _fetch(0, sA, dA, iwA)
    _fetch(1, sB, dB, iwB)
    _fetch_wait(sA, dA, iwA)
    pltpu.async_copy(y_hbm.at[sA.at[0]], rows[0], gs[0])
    pltpu.async_copy(y_hbm.at[sA.at[1]], rows[1], gs[1])

    plsc.subcore_barrier()

    def _pair(i, carry):
        not_last = i < PAIRS - 1
        for phase in range(2):
            swin, dwin = (sA, dA) if phase == 0 else (sB, dB)
            nswin = sB if phase == 0 else sA
            gc_base = tile_chunk0 + (2 * i + phase) * W
            for kk in range(W):
                p = kk % 2
                active = gc_base + kk < NREAL
                if kk == 6:
                    if phase == 0:
                        _fetch_wait(sB, dB, iwB)
                    else:
                        @pl.when(not_last)
                        def _():
                            _fetch_wait(sA, dA, iwA)

                @pl.when(active)
                def _(kk=kk, p=p):
                    pltpu.make_async_copy(
                        y_hbm.at[swin.at[kk]], rows[p], gs[p]).wait()
                    pltpu.async_copy(rows[p], acc_sh.at[dwin.at[kk]], ss[p],
                                     add=True)
                    if with_counts:
                        pltpu.async_copy(ones_v, cnt_sh.at[dwin.at[kk]], cs[p],
                                         add=True)
                    pltpu.make_async_copy(
                        rows[p], acc_sh.at[dwin.at[kk]], ss[p]).wait()

                nxt_active = gc_base + kk + 2 < NREAL
                if kk < W - 2:
                    def _issue(nref=swin, row=rows[p], sem=gs[p], kkn=kk + 2):
                        pltpu.async_copy(y_hbm.at[nref.at[kkn]], row, sem)
                    pl.when(nxt_active)(_issue)
                else:
                    lookahead_ok = (phase == 0) or not_last

                    def _issue(nref=nswin, row=rows[p], sem=gs[p], kkn=kk - 6):
                        pltpu.async_copy(y_hbm.at[nref.at[kkn]], row, sem)
                    if lookahead_ok is True:
                        pl.when(nxt_active)(_issue)
                    else:
                        pl.when(jnp.logical_and(not_last, nxt_active))(_issue)

                if with_counts:
                    @pl.when(active)
                    def _(kk=kk, p=p):
                        pltpu.make_async_copy(
                            ones_v, cnt_sh.at[dwin.at[kk]], cs[p]).wait()
            @pl.when(not_last)
            def _():
                if phase == 0:
                    _fetch(2 * i + 2, sA, dA, iwA)
                else:
                    _fetch(2 * i + 3, sB, dB, iwB)
        return carry
    lax.fori_loop(0, PAIRS, _pair, 0)

    plsc.subcore_barrier()

    pltpu.sync_copy(acc_sh.at[pl.ds(base, RPT)],
                    out_hbm.at[pl.ds(c * NPAD + base, RPT)])
    if with_counts:
        pltpu.sync_copy(cnt_sh.at[pl.ds(base, RPT)],
                        cnt_hbm.at[pl.ds(c * NPAD + base, RPT)])


def _make_sc_kernel(with_counts):
    out_type = [jax.ShapeDtypeStruct((NC * NPAD, D), jnp.float32)]
    if with_counts:
        out_type.append(jax.ShapeDtypeStruct((NC * NPAD,), jnp.float32))
    mesh = plsc.VectorSubcoreMesh(core_axis_name="c", subcore_axis_name="s")
    return pl.kernel(
        functools.partial(_sc_body, with_counts),
        out_type=out_type,
        mesh=mesh,
        scratch_types=[
            pltpu.VMEM((W, CH), jnp.int32),
            pltpu.VMEM((W, CH), jnp.int32),
            pltpu.VMEM((W, CH), jnp.int32),
            pltpu.VMEM((W, CH), jnp.int32),
            pltpu.VMEM((CH, D), jnp.float32),
            pltpu.VMEM((CH, D), jnp.float32),
            pltpu.VMEM((CH,), jnp.float32),
            pltpu.VMEM((RPT,), jnp.float32),
            pltpu.VMEM_SHARED((NPAD, D), jnp.float32),
            pltpu.VMEM_SHARED((NPAD,), jnp.float32),
        ] + [pltpu.SemaphoreType.DMA] * 8,
        name="sage_segment_sum" + ("_cnt" if with_counts else ""),
    )


_sc_layer1 = _make_sc_kernel(True)
_sc_layer2 = _make_sc_kernel(False)


_BLK = 2000


def _dotT(a, w):
    return lax.dot_general(a, w, (((1,), (1,)), ((), ())),
                           preferred_element_type=jnp.float32)


def _pre_body(x_ref, wl_ref, wr_ref, b_ref, y_ref, z_ref):
    xb = x_ref[...]
    y_ref[...] = _dotT(xb, wl_ref[...])
    z_ref[...] = _dotT(xb, wr_ref[...]) + b_ref[...]


def _mid_body(p0_ref, p1_ref, c0_ref, c1_ref, z1_ref, wl_ref, wr_ref, b_ref,
              y2_ref, z2_ref):
    cnt = jnp.maximum(c0_ref[...] + c1_ref[...], 1.0)
    h = jnp.maximum((p0_ref[...] + p1_ref[...]) / cnt + z1_ref[...], 0.0)
    y2_ref[...] = _dotT(h, wl_ref[...])
    z2_ref[...] = _dotT(h, wr_ref[...]) + b_ref[...]


def _post_body(p0_ref, p1_ref, c0_ref, c1_ref, z2_ref, o_ref):
    cnt = jnp.maximum(c0_ref[...] + c1_ref[...], 1.0)
    o_ref[...] = (p0_ref[...] + p1_ref[...]) / cnt + z2_ref[...]


_row_spec = pl.BlockSpec((_BLK, D), lambda i: (i, 0))
_cnt_spec = pl.BlockSpec((_BLK, 1), lambda i: (i, 0))
_w_spec = pl.BlockSpec((D, D), lambda i: (0, 0))
_b_spec = pl.BlockSpec((1, D), lambda i: (0, 0))
_f32 = jnp.float32


_pre_call = pl.pallas_call(
    _pre_body,
    grid=(N // _BLK,),
    in_specs=[_row_spec, _w_spec, _w_spec, _b_spec],
    out_specs=[_row_spec, _row_spec],
    out_shape=[jax.ShapeDtypeStruct((N, D), _f32)] * 2,
)

_mid_call = pl.pallas_call(
    _mid_body,
    grid=(N // _BLK,),
    in_specs=[_row_spec, _row_spec, _cnt_spec, _cnt_spec, _row_spec,
              _w_spec, _w_spec, _b_spec],
    out_specs=[_row_spec, _row_spec],
    out_shape=[jax.ShapeDtypeStruct((N, D), _f32)] * 2,
)

_post_call = pl.pallas_call(
    _post_body,
    grid=(N // _BLK,),
    in_specs=[_row_spec, _row_spec, _cnt_spec, _cnt_spec, _row_spec],
    out_specs=_row_spec,
    out_shape=jax.ShapeDtypeStruct((N, D), _f32),
)


def kernel(x, edge_index, W1l, b1, W1r, W2l, b2, W2r):
    src = edge_index[0].astype(jnp.int32)
    dst = edge_index[1].astype(jnp.int32)
    pad = E_PAD - E
    src_p = jnp.concatenate([src, jnp.zeros((pad,), jnp.int32)]).reshape(E_PAD // CH, CH)
    pad_dst = N + (jnp.arange(pad, dtype=jnp.int32) % (NPAD - N))
    dst_p = jnp.concatenate([dst, pad_dst]).reshape(E_PAD // CH, CH)

    b1r = b1.reshape(1, D)
    b2r = b2.reshape(1, D)

    y1, z1 = _pre_call(x, W1l, W1r, b1r)
    parts1, cnts = _sc_layer1(y1, src_p, dst_p)
    c0 = cnts[0:N].reshape(N, 1)
    c1 = cnts[NPAD:NPAD + N].reshape(N, 1)

    y2, z2 = _mid_call(parts1[0:N], parts1[NPAD:NPAD + N], c0, c1, z1,
                       W2l, W2r, b2r)
    (parts2,) = _sc_layer2(y2, src_p, dst_p)
    return _post_call(parts2[0:N], parts2[NPAD:NPAD + N], c0, c1, z2)


# speedup vs baseline: 3.4847x; 1.0463x over previous
import functools

import jax
import jax.numpy as jnp
from jax import lax
from jax.experimental import pallas as pl
from jax.experimental.pallas import tpu as pltpu
from jax.experimental.pallas import tpu_sc as plsc

N = 10000
E = 320000
D = 128

NC = 2
NS = 16
NW = NC * NS

CH = 128
NCH = 80
E_PAD = NW * CH * NCH
NREAL = E // CH
NPAD = 10240
RPT = NPAD // NS


W = 8
PAIRS = NCH // (2 * W)


def _sc_body(with_counts, *refs):
    if with_counts:
        (y_hbm, src_hbm, dst_hbm, out_hbm, cnt_hbm,
         sA, dA, sB, dB, rows0, rows1, ones_v, zc_v,
         acc_sh, cnt_sh, *sems) = refs
    else:
        (y_hbm, src_hbm, dst_hbm, out_hbm,
         sA, dA, sB, dB, rows0, rows1, ones_v, zc_v,
         acc_sh, cnt_sh, *sems) = refs
    rows = (rows0, rows1)
    gs0, gs1, ss0, ss1, cs0, cs1, iwA, iwB = sems
    gs = (gs0, gs1)
    ss = (ss0, ss1)
    cs = (cs0, cs1)

    c = lax.axis_index("c")
    s = lax.axis_index("s")
    base = s * RPT
    wid = c * NS + s
    tile_chunk0 = wid * NCH

    zeros16 = jnp.zeros((16,), jnp.float32)

    def _zrow(i, carry):
        for j in range(D // 16):
            rows0[i, pl.ds(j * 16, 16)] = zeros16
        return carry
    lax.fori_loop(0, CH, _zrow, 0)
    for j in range(CH // 16):
        ones_v[pl.ds(j * 16, 16)] = jnp.ones((16,), jnp.float32)
    for j in range(RPT // 16):
        zc_v[pl.ds(j * 16, 16)] = zeros16

    for k in range(RPT // CH):
        pltpu.sync_copy(rows0, acc_sh.at[pl.ds(base + k * CH, CH)])
    pltpu.sync_copy(zc_v, cnt_sh.at[pl.ds(base, RPT)])

    def _fetch(win_idx, sbuf, dbuf, sem):
        off = tile_chunk0 + win_idx * W
        pltpu.async_copy(src_hbm.at[pl.ds(off, W)], sbuf, sem)
        pltpu.async_copy(dst_hbm.at[pl.ds(off, W)], dbuf, sem)

    def _fetch_wait(sbuf, dbuf, sem):
        pltpu.make_async_copy(src_hbm.at[pl.ds(0, W)], sbuf, sem).wait()
        pltpu.make_async_copy(dst_hbm.at[pl.ds(0, W)], dbuf, sem).wait()

    _fetch(0, sA, dA, iwA)
    _fetch(1, sB, dB, iwB)
    _fetch_wait(sA, dA, iwA)
    pltpu.async_copy(y_hbm.at[sA.at[0]], rows[0], gs[0])
    pltpu.async_copy(y_hbm.at[sA.at[1]], rows[1], gs[1])

    plsc.subcore_barrier()

    def _pair(i, carry):
        not_last = i < PAIRS - 1
        for phase in range(2):
            swin, dwin = (sA, dA) if phase == 0 else (sB, dB)
            nswin = sB if phase == 0 else sA
            gc_base = tile_chunk0 + (2 * i + phase) * W
            for kk in range(W):
                p = kk % 2
                active = gc_base + kk < NREAL
                if kk == 6:
                    if phase == 0:
                        _fetch_wait(sB, dB, iwB)
                    else:
                        @pl.when(not_last)
                        def _():
                            _fetch_wait(sA, dA, iwA)

                @pl.when(active)
                def _(kk=kk, p=p):
                    pltpu.make_async_copy(
                        y_hbm.at[swin.at[kk]], rows[p], gs[p]).wait()
                    pltpu.async_copy(rows[p], acc_sh.at[dwin.at[kk]], ss[p],
                                     add=True)
                    if with_counts:
                        pltpu.async_copy(ones_v, cnt_sh.at[dwin.at[kk]], cs[p],
                                         add=True)
                    pltpu.make_async_copy(
                        rows[p], acc_sh.at[dwin.at[kk]], ss[p]).wait()

                nxt_active = gc_base + kk + 2 < NREAL
                if kk < W - 2:
                    def _issue(nref=swin, row=rows[p], sem=gs[p], kkn=kk + 2):
                        pltpu.async_copy(y_hbm.at[nref.at[kkn]], row, sem)
                    pl.when(nxt_active)(_issue)
                else:
                    lookahead_ok = (phase == 0) or not_last

                    def _issue(nref=nswin, row=rows[p], sem=gs[p], kkn=kk - 6):
                        pltpu.async_copy(y_hbm.at[nref.at[kkn]], row, sem)
                    if lookahead_ok is True:
                        pl.when(nxt_active)(_issue)
                    else:
                        pl.when(jnp.logical_and(not_last, nxt_active))(_issue)

                if with_counts:
                    @pl.when(active)
                    def _(kk=kk, p=p):
                        pltpu.make_async_copy(
                            ones_v, cnt_sh.at[dwin.at[kk]], cs[p]).wait()
            @pl.when(not_last)
            def _():
                if phase == 0:
                    _fetch(2 * i + 2, sA, dA, iwA)
                else:
                    _fetch(2 * i + 3, sB, dB, iwB)
        return carry
    lax.fori_loop(0, PAIRS, _pair, 0)

    plsc.subcore_barrier()

    RLAST = N - (NS - 1) * RPT

    @pl.when(s < NS - 1)
    def _():
        pltpu.sync_copy(acc_sh.at[pl.ds(base, RPT)],
                        out_hbm.at[pl.ds(c * N + base, RPT)])

    @pl.when(s == NS - 1)
    def _():
        pltpu.sync_copy(acc_sh.at[pl.ds(base, RLAST)],
                        out_hbm.at[pl.ds(c * N + base, RLAST)])
    if with_counts:
        pltpu.sync_copy(cnt_sh.at[pl.ds(base, RPT)],
                        cnt_hbm.at[pl.ds(c * NPAD + base, RPT)])


def _make_sc_kernel(with_counts):
    out_type = [jax.ShapeDtypeStruct((NC * N, D), jnp.float32)]
    if with_counts:
        out_type.append(jax.ShapeDtypeStruct((NC * NPAD,), jnp.float32))
    mesh = plsc.VectorSubcoreMesh(core_axis_name="c", subcore_axis_name="s")
    return pl.kernel(
        functools.partial(_sc_body, with_counts),
        out_type=out_type,
        mesh=mesh,
        scratch_types=[
            pltpu.VMEM((W, CH), jnp.int32),
            pltpu.VMEM((W, CH), jnp.int32),
            pltpu.VMEM((W, CH), jnp.int32),
            pltpu.VMEM((W, CH), jnp.int32),
            pltpu.VMEM((CH, D), jnp.float32),
            pltpu.VMEM((CH, D), jnp.float32),
            pltpu.VMEM((CH,), jnp.float32),
            pltpu.VMEM((RPT,), jnp.float32),
            pltpu.VMEM_SHARED((NPAD, D), jnp.float32),
            pltpu.VMEM_SHARED((NPAD,), jnp.float32),
        ] + [pltpu.SemaphoreType.DMA] * 8,
        name="sage_segment_sum" + ("_cnt" if with_counts else ""),
    )


_sc_layer1 = _make_sc_kernel(True)
_sc_layer2 = _make_sc_kernel(False)


_BLK = 2000


def _dotT(a, w):
    return lax.dot_general(a, w, (((1,), (1,)), ((), ())),
                           preferred_element_type=jnp.float32)


def _pre_body(x_ref, wl_ref, wr_ref, b_ref, y_ref, z_ref):
    xb = x_ref[...]
    y_ref[...] = _dotT(xb, wl_ref[...])
    z_ref[...] = _dotT(xb, wr_ref[...]) + b_ref[...]


def _mid_body(p0_ref, p1_ref, c0_ref, c1_ref, z1_ref, wl_ref, wr_ref, b_ref,
              y2_ref, z2_ref):
    cnt = jnp.maximum(c0_ref[...] + c1_ref[...], 1.0)
    h = jnp.maximum((p0_ref[...] + p1_ref[...]) / cnt + z1_ref[...], 0.0)
    y2_ref[...] = _dotT(h, wl_ref[...])
    z2_ref[...] = _dotT(h, wr_ref[...]) + b_ref[...]


def _post_body(p0_ref, p1_ref, c0_ref, c1_ref, z2_ref, o_ref):
    cnt = jnp.maximum(c0_ref[...] + c1_ref[...], 1.0)
    o_ref[...] = (p0_ref[...] + p1_ref[...]) / cnt + z2_ref[...]


_row_spec = pl.BlockSpec((_BLK, D), lambda i: (i, 0))
_row1_spec = pl.BlockSpec((_BLK, D), lambda i: (i + N // _BLK, 0))
_cnt_spec = pl.BlockSpec((_BLK, 1), lambda i: (i, 0))
_cnt1_spec = pl.BlockSpec((_BLK, 1), lambda i: (i + N // _BLK, 0))
_w_spec = pl.BlockSpec((D, D), lambda i: (0, 0))
_b_spec = pl.BlockSpec((1, D), lambda i: (0, 0))
_f32 = jnp.float32


_pre_call = pl.pallas_call(
    _pre_body,
    grid=(N // _BLK,),
    in_specs=[_row_spec, _w_spec, _w_spec, _b_spec],
    out_specs=[_row_spec, _row_spec],
    out_shape=[jax.ShapeDtypeStruct((N, D), _f32)] * 2,
)

_mid_call = pl.pallas_call(
    _mid_body,
    grid=(N // _BLK,),
    in_specs=[_row_spec, _row1_spec, _cnt_spec, _cnt1_spec, _row_spec,
              _w_spec, _w_spec, _b_spec],
    out_specs=[_row_spec, _row_spec],
    out_shape=[jax.ShapeDtypeStruct((N, D), _f32)] * 2,
)

_post_call = pl.pallas_call(
    _post_body,
    grid=(N // _BLK,),
    in_specs=[_row_spec, _row1_spec, _cnt_spec, _cnt1_spec, _row_spec],
    out_specs=_row_spec,
    out_shape=jax.ShapeDtypeStruct((N, D), _f32),
)


def kernel(x, edge_index, W1l, b1, W1r, W2l, b2, W2r):
    src = edge_index[0].astype(jnp.int32)
    dst = edge_index[1].astype(jnp.int32)
    pad = E_PAD - E
    src_p = jnp.concatenate([src, jnp.zeros((pad,), jnp.int32)]).reshape(E_PAD // CH, CH)
    pad_dst = N + (jnp.arange(pad, dtype=jnp.int32) % (NPAD - N))
    dst_p = jnp.concatenate([dst, pad_dst]).reshape(E_PAD // CH, CH)

    b1r = b1.reshape(1, D)
    b2r = b2.reshape(1, D)

    y1, z1 = _pre_call(x, W1l, W1r, b1r)
    parts1, cnts = _sc_layer1(y1, src_p, dst_p)
    c0 = cnts[0:N].reshape(N, 1)
    c1 = cnts[NPAD:NPAD + N].reshape(N, 1)

    y2, z2 = _mid_call(parts1, parts1, c0, c1, z1, W2l, W2r, b2r)
    (parts2,) = _sc_layer2(y2, src_p, dst_p)
    return _post_call(parts2, parts2, c0, c1, z2)


# speedup vs baseline: 3.4964x; 1.0034x over previous
import functools

import jax
import jax.numpy as jnp
from jax import lax
from jax.experimental import pallas as pl
from jax.experimental.pallas import tpu as pltpu
from jax.experimental.pallas import tpu_sc as plsc

N = 10000
E = 320000
D = 128

NC = 2
NS = 16
NW = NC * NS

CH = 128
NCH = 80
E_PAD = NW * CH * NCH
NREAL = E // CH
NPAD = 10240
RPT = NPAD // NS


W = 8
PAIRS = NCH // (2 * W)


def _sc_body(with_counts, *refs):
    if with_counts:
        (y_hbm, src_hbm, dst_hbm, out_hbm, cnt_hbm,
         sA, dA, sB, dB, rows0, rows1, ones_v, zc_v,
         acc_sh, cnt_sh, *sems) = refs
    else:
        (y_hbm, src_hbm, dst_hbm, out_hbm,
         sA, dA, sB, dB, rows0, rows1, ones_v, zc_v,
         acc_sh, cnt_sh, *sems) = refs
    rows = (rows0, rows1)
    gs0, gs1, ss0, ss1, cs0, cs1, iwA, iwB = sems
    gs = (gs0, gs1)
    ss = (ss0, ss1)
    cs = (cs0, cs1)

    c = lax.axis_index("c")
    s = lax.axis_index("s")
    base = s * RPT
    wid = c * NS + s
    tile_chunk0 = wid * NCH

    zeros16 = jnp.zeros((16,), jnp.float32)

    def _zrow(i, carry):
        for j in range(D // 16):
            rows0[i, pl.ds(j * 16, 16)] = zeros16
        return carry
    lax.fori_loop(0, CH, _zrow, 0)
    for j in range(CH // 16):
        ones_v[pl.ds(j * 16, 16)] = jnp.ones((16,), jnp.float32)
    for j in range(RPT // 16):
        zc_v[pl.ds(j * 16, 16)] = zeros16

    for k in range(RPT // CH):
        pltpu.sync_copy(rows0, acc_sh.at[pl.ds(base + k * CH, CH)])
    pltpu.sync_copy(zc_v, cnt_sh.at[pl.ds(base, RPT)])

    def _fetch(win_idx, sbuf, dbuf, sem):
        off = tile_chunk0 + win_idx * W
        pltpu.async_copy(src_hbm.at[pl.ds(off, W)], sbuf, sem)
        pltpu.async_copy(dst_hbm.at[pl.ds(off, W)], dbuf, sem)

    def _fetch_wait(sbuf, dbuf, sem):
        pltpu.make_async_copy(src_hbm.at[pl.ds(0, W)], sbuf, sem).wait()
        pltpu.make_async_copy(dst_hbm.at[pl.ds(0, W)], dbuf, sem).wait()

    _fetch(0, sA, dA, iwA)
    _fetch(1, sB, dB, iwB)
    _fetch_wait(sA, dA, iwA)
    pltpu.async_copy(y_hbm.at[sA.at[0]], rows[0], gs[0])
    pltpu.async_copy(y_hbm.at[sA.at[1]], rows[1], gs[1])

    plsc.subcore_barrier()

    def _pair(i, carry):
        not_last = i < PAIRS - 1
        for phase in range(2):
            swin, dwin = (sA, dA) if phase == 0 else (sB, dB)
            nswin = sB if phase == 0 else sA
            gc_base = tile_chunk0 + (2 * i + phase) * W
            for kk in range(W):
                p = kk % 2
                active = gc_base + kk < NREAL
                if kk == 6:
                    if phase == 0:
                        _fetch_wait(sB, dB, iwB)
                    else:
                        @pl.when(not_last)
                        def _():
                            _fetch_wait(sA, dA, iwA)

                @pl.when(active)
                def _(kk=kk, p=p):
                    pltpu.make_async_copy(
                        y_hbm.at[swin.at[kk]], rows[p], gs[p]).wait()
                    pltpu.async_copy(rows[p], acc_sh.at[dwin.at[kk]], ss[p],
                                     add=True)
                    if with_counts:
                        pltpu.async_copy(ones_v, cnt_sh.at[dwin.at[kk]], cs[p],
                                         add=True)
                    pltpu.make_async_copy(
                        rows[p], acc_sh.at[dwin.at[kk]], ss[p]).wait()

                nxt_active = gc_base + kk + 2 < NREAL
                if kk < W - 2:
                    def _issue(nref=swin, row=rows[p], sem=gs[p], kkn=kk + 2):
                        pltpu.async_copy(y_hbm.at[nref.at[kkn]], row, sem)
                    pl.when(nxt_active)(_issue)
                else:
                    lookahead_ok = (phase == 0) or not_last

                    def _issue(nref=nswin, row=rows[p], sem=gs[p], kkn=kk - 6):
                        pltpu.async_copy(y_hbm.at[nref.at[kkn]], row, sem)
                    if lookahead_ok is True:
                        pl.when(nxt_active)(_issue)
                    else:
                        pl.when(jnp.logical_and(not_last, nxt_active))(_issue)

                if with_counts:
                    @pl.when(active)
                    def _(kk=kk, p=p):
                        pltpu.make_async_copy(
                            ones_v, cnt_sh.at[dwin.at[kk]], cs[p]).wait()
            @pl.when(not_last)
            def _():
                if phase == 0:
                    _fetch(2 * i + 2, sA, dA, iwA)
                else:
                    _fetch(2 * i + 3, sB, dB, iwB)
        return carry
    lax.fori_loop(0, PAIRS, _pair, 0)

    plsc.subcore_barrier()

    RLAST = N - (NS - 1) * RPT

    @pl.when(s < NS - 1)
    def _():
        pltpu.sync_copy(acc_sh.at[pl.ds(base, RPT)],
                        out_hbm.at[pl.ds(c * N + base, RPT)])

    @pl.when(s == NS - 1)
    def _():
        pltpu.sync_copy(acc_sh.at[pl.ds(base, RLAST)],
                        out_hbm.at[pl.ds(c * N + base, RLAST)])
    if with_counts:
        pltpu.sync_copy(cnt_sh.at[pl.ds(base, RPT)],
                        cnt_hbm.at[pl.ds(c * NPAD + base, RPT)])


def _make_sc_kernel(with_counts):
    out_type = [jax.ShapeDtypeStruct((NC * N, D), jnp.float32)]
    if with_counts:
        out_type.append(jax.ShapeDtypeStruct((NC * NPAD,), jnp.float32))
    mesh = plsc.VectorSubcoreMesh(core_axis_name="c", subcore_axis_name="s")
    return pl.kernel(
        functools.partial(_sc_body, with_counts),
        out_type=out_type,
        mesh=mesh,
        scratch_types=[
            pltpu.VMEM((W, CH), jnp.int32),
            pltpu.VMEM((W, CH), jnp.int32),
            pltpu.VMEM((W, CH), jnp.int32),
            pltpu.VMEM((W, CH), jnp.int32),
            pltpu.VMEM((CH, D), jnp.float32),
            pltpu.VMEM((CH, D), jnp.float32),
            pltpu.VMEM((CH,), jnp.float32),
            pltpu.VMEM((RPT,), jnp.float32),
            pltpu.VMEM_SHARED((NPAD, D), jnp.float32),
            pltpu.VMEM_SHARED((NPAD,), jnp.float32),
        ] + [pltpu.SemaphoreType.DMA] * 8,
        name="sage_segment_sum" + ("_cnt" if with_counts else ""),
    )


_sc_layer1 = _make_sc_kernel(True)
_sc_layer2 = _make_sc_kernel(False)


_BLK = 2000


def _dotT(a, w):
    return lax.dot_general(a, w, (((1,), (1,)), ((), ())),
                           preferred_element_type=jnp.float32)


def _pre_body(x_ref, wl_ref, wr_ref, b_ref, y_ref, z_ref):
    xb = x_ref[...]
    y_ref[...] = _dotT(xb, wl_ref[...])
    z_ref[...] = _dotT(xb, wr_ref[...]) + b_ref[...]


def _mid_body(p0_ref, p1_ref, c0_ref, c1_ref, z1_ref, wl_ref, wr_ref, b_ref,
              y2_ref, z2_ref):
    cnt = jnp.maximum(c0_ref[...] + c1_ref[...], 1.0)
    h = jnp.maximum((p0_ref[...] + p1_ref[...]) / cnt + z1_ref[...], 0.0)
    y2_ref[...] = _dotT(h, wl_ref[...])
    z2_ref[...] = _dotT(h, wr_ref[...]) + b_ref[...]


def _post_body(p0_ref, p1_ref, c0_ref, c1_ref, z2_ref, o_ref):
    cnt = jnp.maximum(c0_ref[...] + c1_ref[...], 1.0)
    o_ref[...] = (p0_ref[...] + p1_ref[...]) / cnt + z2_ref[...]


_row_spec = pl.BlockSpec((_BLK, D), lambda i: (i, 0))
_row1_spec = pl.BlockSpec((_BLK, D), lambda i: (i + N // _BLK, 0))
_cnt_spec = pl.BlockSpec((_BLK, 1), lambda i: (i, 0))
_w_spec = pl.BlockSpec((D, D), lambda i: (0, 0))
_b_spec = pl.BlockSpec((1, D), lambda i: (0, 0))
_f32 = jnp.float32


_pre_call = pl.pallas_call(
    _pre_body,
    grid=(N // _BLK,),
    in_specs=[_row_spec, _w_spec, _w_spec, _b_spec],
    out_specs=[_row_spec, _row_spec],
    out_shape=[jax.ShapeDtypeStruct((N, D), _f32)] * 2,
)

_mid_call = pl.pallas_call(
    _mid_body,
    grid=(N // _BLK,),
    in_specs=[_row_spec, _row1_spec, _cnt_spec, _cnt_spec, _row_spec,
              _w_spec, _w_spec, _b_spec],
    out_specs=[_row_spec, _row_spec],
    out_shape=[jax.ShapeDtypeStruct((N, D), _f32)] * 2,
)

_post_call = pl.pallas_call(
    _post_body,
    grid=(N // _BLK,),
    in_specs=[_row_spec, _row1_spec, _cnt_spec, _cnt_spec, _row_spec],
    out_specs=_row_spec,
    out_shape=jax.ShapeDtypeStruct((N, D), _f32),
)


def kernel(x, edge_index, W1l, b1, W1r, W2l, b2, W2r):
    src = edge_index[0].astype(jnp.int32)
    dst = edge_index[1].astype(jnp.int32)
    pad = E_PAD - E
    src_p = jnp.concatenate([src, jnp.zeros((pad,), jnp.int32)]).reshape(E_PAD // CH, CH)
    pad_dst = N + (jnp.arange(pad, dtype=jnp.int32) % (NPAD - N))
    dst_p = jnp.concatenate([dst, pad_dst]).reshape(E_PAD // CH, CH)

    b1r = b1.reshape(1, D)
    b2r = b2.reshape(1, D)

    y1, z1 = _pre_call(x, W1l, W1r, b1r)
    parts1, cnts = _sc_layer1(y1, src_p, dst_p)
    c0 = cnts[0:N].reshape(N, 1)
    c1 = cnts[NPAD:NPAD + N].reshape(N, 1)

    y2, z2 = _mid_call(parts1, parts1, c0, c1, z1, W2l, W2r, b2r)
    (parts2,) = _sc_layer2(y2, src_p, dst_p)
    return _post_call(parts2, parts2, c0, c1, z2)
